# Initial kernel scaffold; baseline (speedup 1.0000x reference)
import jax
import jax.numpy as jnp
from jax.experimental import pallas as pl


def kernel(X, part_masks, all_h_t, all_c_t, Y, T_obs, T_pred, W_in, b_in, W_soc, b_soc, W_ih, W_hh, b_ih, b_hh, W_out, b_out):
    raise NotImplementedError("write your pallas kernel here")


# speedup vs baseline: 1.6034x; 1.6034x over previous
import functools

import jax
import jax.numpy as jnp
from jax import lax
from jax.experimental import pallas as pl
from jax.experimental.pallas import tpu as pltpu
from jax.experimental.pallas import tpu_sc as plsc

T_ALL, N = 20, 4096
NSTEP = 19
INPUT_DIM, HIDDEN, MEDIATE, OUT_DIM, SOCIAL = 2, 128, 128, 2, 16
N_SIZE = 2
CELL = 0.3
GRID = 256
PAD = 8

NC, NS = 2, 16
NW = NC * NS
ROWS_PER_TILE = 257
R_TAB = NS * ROWS_PER_TILE
DUMMY = N
SC_SCAT = N // NS
SC_GATH = N // NW


def _precompute_indices(X, part_masks):
    Xs = X[:NSTEP]
    m = part_masks[:NSTEP]
    margin = 2 * N_SIZE * CELL
    x_min = Xs[:, :, 0].min(axis=1, keepdims=True) - margin
    y_min = Xs[:, :, 1].min(axis=1, keepdims=True) - margin
    px = jnp.floor((Xs[:, :, 0] - x_min) / CELL).astype(jnp.int32) * m.astype(jnp.int32)
    py = jnp.floor((Xs[:, :, 1] - y_min) / CELL).astype(jnp.int32) * m.astype(jnp.int32)
    px = jnp.clip(px, 0, GRID - 1)
    py = jnp.clip(py, 0, GRID - 1)
    key = px * GRID + py
    g = key - GRID - 1
    order = jnp.argsort(key, axis=1)
    sk = jnp.take_along_axis(key, order, axis=1)
    newseg = jnp.concatenate(
        [jnp.ones((NSTEP, 1), bool), sk[:, 1:] != sk[:, :-1]], axis=1)
    cid_sorted = jnp.cumsum(newseg.astype(jnp.int32), axis=1) - 1
    rows = jnp.arange(NSTEP)[:, None]
    cid = jnp.zeros((NSTEP, N), jnp.int32).at[rows, order].set(cid_sorted)
    sent = jnp.int32(2 ** 30)
    uniq = jnp.full((NSTEP, N), sent, jnp.int32).at[rows, cid_sorted].set(sk)
    pos = jax.vmap(jnp.searchsorted)(uniq, g).astype(jnp.int32)
    posc = jnp.clip(pos, 0, N - 1)
    found = (jnp.take_along_axis(uniq, posc, axis=1) == g) & (m != 0)
    gid = jnp.where(found, posc, jnp.int32(DUMMY))
    return cid.reshape(NSTEP, N // 128, 128), gid.reshape(NSTEP, N // 128, 128)


def _pool_body(h_hbm, cid_hbm, gid_hbm, zeros_hbm, vals_hbm,
               h_buf, vals_buf, cidx, gidx, table, sem):
    c = lax.axis_index("c")
    s = lax.axis_index("s")
    wid = c * NS + s
    pltpu.sync_copy(zeros_hbm, table.at[pl.ds(s * ROWS_PER_TILE, ROWS_PER_TILE)])
    pltpu.sync_copy(h_hbm.at[pl.ds(s * SC_SCAT, SC_SCAT)], h_buf)
    pltpu.sync_copy(cid_hbm.at[pl.ds(s * 2, 2)], cidx)
    pltpu.sync_copy(gid_hbm.at[pl.ds(wid, 1)], gidx)
    plsc.subcore_barrier()
    pltpu.sync_copy(h_buf.at[pl.ds(0, 128)], table.at[cidx.at[0]], add=True)
    pltpu.sync_copy(h_buf.at[pl.ds(128, 128)], table.at[cidx.at[1]], add=True)
    plsc.subcore_barrier()
    pltpu.async_copy(table.at[gidx.at[0]], vals_buf, sem).wait()
    pltpu.sync_copy(vals_buf, vals_hbm.at[pl.ds(wid * SC_GATH, SC_GATH)])


@functools.cache
def _get_pool():
    return pl.kernel(
        _pool_body,
        out_type=jax.ShapeDtypeStruct((N, HIDDEN), jnp.float32),
        mesh=plsc.VectorSubcoreMesh(core_axis_name="c", subcore_axis_name="s",
                                    num_cores=NC, num_subcores=NS),
        scratch_types=[
            pltpu.VMEM((SC_SCAT, HIDDEN), jnp.float32),
            pltpu.VMEM((SC_GATH, HIDDEN), jnp.float32),
            pltpu.VMEM((2, 128), jnp.int32),
            pltpu.VMEM((1, 128), jnp.int32),
            pltpu.VMEM_SHARED((R_TAB, HIDDEN), jnp.float32),
            pltpu.SemaphoreType.DMA,
        ],
        name="social_pool_sc",
    )


_BLK = 1024


def _step_body(inp_ref, h_ref, c_ref, vals_ref, winT, wsT, wrT, weT, whhT,
               bg, bi, bs, bo, woT, h2_ref, c2_ref, out_ref):
    f32 = jnp.float32
    r = jnp.maximum(
        jnp.dot(inp_ref[...], winT[...], preferred_element_type=f32) + bi[0:1, :], 0.0)
    e = jnp.maximum(
        jnp.dot(vals_ref[...], wsT[...], preferred_element_type=f32) + bs[0:1, :], 0.0)
    g = (jnp.dot(r, wrT[...], preferred_element_type=f32)
         + jnp.dot(e, weT[...], preferred_element_type=f32)
         + jnp.dot(h_ref[...], whhT[...], preferred_element_type=f32)
         + bg[0:1, :])
    ii = jax.nn.sigmoid(g[:, 0:HIDDEN])
    ff = jax.nn.sigmoid(g[:, HIDDEN:2 * HIDDEN])
    gg = jnp.tanh(g[:, 2 * HIDDEN:3 * HIDDEN])
    oo = jax.nn.sigmoid(g[:, 3 * HIDDEN:4 * HIDDEN])
    c2 = ff * c_ref[...] + ii * gg
    h2 = oo * jnp.tanh(c2)
    h2_ref[...] = h2
    c2_ref[...] = c2
    out_ref[...] = jnp.dot(h2, woT[...], preferred_element_type=f32) + bo[0:1, :]


def _make_tc_step():
    data = lambda w: pl.BlockSpec((_BLK, w), lambda i: (i, 0))
    full = lambda a, b: pl.BlockSpec((a, b), lambda i: (0, 0))
    return pl.pallas_call(
        _step_body,
        grid=(N // _BLK,),
        in_specs=[
            data(PAD), data(HIDDEN), data(HIDDEN), data(HIDDEN),
            full(PAD, HIDDEN), full(HIDDEN, SOCIAL), full(HIDDEN, 4 * HIDDEN),
            full(SOCIAL, 4 * HIDDEN), full(HIDDEN, 4 * HIDDEN),
            full(PAD, 4 * HIDDEN), full(PAD, HIDDEN), full(PAD, SOCIAL),
            full(PAD, PAD), full(HIDDEN, PAD),
        ],
        out_specs=[data(HIDDEN), data(HIDDEN), data(PAD)],
        out_shape=[
            jax.ShapeDtypeStruct((N, HIDDEN), jnp.float32),
            jax.ShapeDtypeStruct((N, HIDDEN), jnp.float32),
            jax.ShapeDtypeStruct((N, PAD), jnp.float32),
        ],
        name="social_lstm_step_tc",
    )


_tc_step = functools.cache(_make_tc_step)


def kernel(X, part_masks, all_h_t, all_c_t, Y, T_obs, T_pred, W_in, b_in,
           W_soc, b_soc, W_ih, W_hh, b_ih, b_hh, W_out, b_out):
    f32 = jnp.float32
    cid, gid = _precompute_indices(X, part_masks)
    zeros_tab = jnp.zeros((ROWS_PER_TILE, HIDDEN), f32)

    w_soc_eff = W_soc.reshape(SOCIAL, (N_SIZE + 1) ** 2, HIDDEN).sum(axis=1)
    winT = jnp.zeros((PAD, HIDDEN), f32).at[:INPUT_DIM].set(W_in.T)
    wsT = w_soc_eff.T
    wrT = W_ih[:, :MEDIATE].T
    weT = W_ih[:, MEDIATE:].T
    whhT = W_hh.T
    bg = jnp.broadcast_to(b_ih + b_hh, (PAD, 4 * HIDDEN))
    bi = jnp.broadcast_to(b_in, (PAD, HIDDEN))
    bs = jnp.broadcast_to(b_soc, (PAD, SOCIAL))
    bo = jnp.zeros((PAD, PAD), f32).at[:, :OUT_DIM].set(
        jnp.broadcast_to(b_out, (PAD, OUT_DIM)))
    woT = jnp.zeros((HIDDEN, PAD), f32).at[:, :OUT_DIM].set(W_out.T)

    Xp = jnp.zeros((NSTEP, N, PAD), f32).at[:, :, :INPUT_DIM].set(X[:NSTEP])

    h, c = all_h_t, all_c_t
    outs = []
    for t in range(NSTEP):
        inp = Xp[min(t, 9)] if t <= 9 else outs[t - 2]
        vals = _get_pool()(h, cid[t], gid[t], zeros_tab)
        h, c, out = _tc_step()(inp, h, c, vals, winT, wsT, wrT, weT, whhT,
                               bg, bi, bs, bo, woT)
        outs.append(out)
    outs.append(jnp.zeros((N, PAD), f32))
    res = jnp.stack(outs, axis=0)[:, :, :OUT_DIM]
    return res * part_masks[:, :, None]


# speedup vs baseline: 4.0913x; 2.5516x over previous
import functools

import jax
import jax.numpy as jnp
from jax import lax
from jax.experimental import pallas as pl
from jax.experimental.pallas import tpu as pltpu
from jax.experimental.pallas import tpu_sc as plsc

T_ALL, N = 20, 4096
NSTEP = 19
INPUT_DIM, HIDDEN, MEDIATE, OUT_DIM, SOCIAL = 2, 128, 128, 2, 16
N_SIZE = 2
CELL = 0.3
GRID = 256
PAD = 8

NC, NS = 2, 16
NW = NC * NS
ROWS_PER_TILE = 257
R_TAB = NS * ROWS_PER_TILE
DUMMY = N
SC_SCAT = N // NS
SC_GATH = N // NW


def _precompute_indices(X, part_masks):
    Xs = X[:NSTEP]
    m = part_masks[:NSTEP]
    margin = 2 * N_SIZE * CELL
    x_min = Xs[:, :, 0].min(axis=1, keepdims=True) - margin
    y_min = Xs[:, :, 1].min(axis=1, keepdims=True) - margin
    px = jnp.floor((Xs[:, :, 0] - x_min) / CELL).astype(jnp.int32) * m.astype(jnp.int32)
    py = jnp.floor((Xs[:, :, 1] - y_min) / CELL).astype(jnp.int32) * m.astype(jnp.int32)
    px = jnp.clip(px, 0, GRID - 1)
    py = jnp.clip(py, 0, GRID - 1)
    key = px * GRID + py
    g = jnp.clip(key - GRID - 1, 0, GRID * GRID - 1)
    sent = jnp.int32(2 ** 30)
    rows = jnp.arange(NSTEP)[:, None]
    ids = jnp.broadcast_to(jnp.arange(N, dtype=jnp.int32), (NSTEP, N))
    lut = jnp.full((NSTEP, GRID * GRID), sent, jnp.int32).at[rows, key].min(ids)
    cid = jnp.take_along_axis(lut, key, axis=1)
    gr = jnp.take_along_axis(lut, g, axis=1)
    found = (gr < sent) & (m != 0) & (key - GRID - 1 >= 0)
    gid = jnp.where(found, gr, jnp.int32(DUMMY))
    return cid.reshape(NSTEP, N // 128, 128), gid.reshape(NSTEP, N // 128, 128)


def _pool_body(h_hbm, cid_hbm, gid_hbm, zeros_hbm, vals_hbm,
               h_buf, vals_buf, cidx, gidx, table, sem):
    c = lax.axis_index("c")
    s = lax.axis_index("s")
    wid = c * NS + s
    pltpu.sync_copy(zeros_hbm, table.at[pl.ds(s * ROWS_PER_TILE, ROWS_PER_TILE)])
    pltpu.sync_copy(h_hbm.at[pl.ds(s * SC_SCAT, SC_SCAT)], h_buf)
    pltpu.sync_copy(cid_hbm.at[pl.ds(s * 2, 2)], cidx)
    pltpu.sync_copy(gid_hbm.at[pl.ds(wid, 1)], gidx)
    plsc.subcore_barrier()
    pltpu.sync_copy(h_buf.at[pl.ds(0, 128)], table.at[cidx.at[0]], add=True)
    pltpu.sync_copy(h_buf.at[pl.ds(128, 128)], table.at[cidx.at[1]], add=True)
    plsc.subcore_barrier()
    pltpu.async_copy(table.at[gidx.at[0]], vals_buf, sem).wait()
    pltpu.sync_copy(vals_buf, vals_hbm.at[pl.ds(wid * SC_GATH, SC_GATH)])


@functools.cache
def _get_pool():
    return pl.kernel(
        _pool_body,
        out_type=jax.ShapeDtypeStruct((N, HIDDEN), jnp.float32),
        mesh=plsc.VectorSubcoreMesh(core_axis_name="c", subcore_axis_name="s",
                                    num_cores=NC, num_subcores=NS),
        scratch_types=[
            pltpu.VMEM((SC_SCAT, HIDDEN), jnp.float32),
            pltpu.VMEM((SC_GATH, HIDDEN), jnp.float32),
            pltpu.VMEM((2, 128), jnp.int32),
            pltpu.VMEM((1, 128), jnp.int32),
            pltpu.VMEM_SHARED((R_TAB, HIDDEN), jnp.float32),
            pltpu.SemaphoreType.DMA,
        ],
        name="social_pool_sc",
    )


_BLK = 1024


def _step_body(inp_ref, h_ref, c_ref, vals_ref, winT, wsT, wrT, weT, whhT,
               bg, bi, bs, bo, woT, h2_ref, c2_ref, out_ref):
    f32 = jnp.float32
    r = jnp.maximum(
        jnp.dot(inp_ref[...], winT[...], preferred_element_type=f32) + bi[0:1, :], 0.0)
    e = jnp.maximum(
        jnp.dot(vals_ref[...], wsT[...], preferred_element_type=f32) + bs[0:1, :], 0.0)
    g = (jnp.dot(r, wrT[...], preferred_element_type=f32)
         + jnp.dot(e, weT[...], preferred_element_type=f32)
         + jnp.dot(h_ref[...], whhT[...], preferred_element_type=f32)
         + bg[0:1, :])
    ii = jax.nn.sigmoid(g[:, 0:HIDDEN])
    ff = jax.nn.sigmoid(g[:, HIDDEN:2 * HIDDEN])
    gg = jnp.tanh(g[:, 2 * HIDDEN:3 * HIDDEN])
    oo = jax.nn.sigmoid(g[:, 3 * HIDDEN:4 * HIDDEN])
    c2 = ff * c_ref[...] + ii * gg
    h2 = oo * jnp.tanh(c2)
    h2_ref[...] = h2
    c2_ref[...] = c2
    out_ref[...] = jnp.dot(h2, woT[...], preferred_element_type=f32) + bo[0:1, :]


def _make_tc_step():
    data = lambda w: pl.BlockSpec((_BLK, w), lambda i: (i, 0))
    full = lambda a, b: pl.BlockSpec((a, b), lambda i: (0, 0))
    return pl.pallas_call(
        _step_body,
        grid=(N // _BLK,),
        in_specs=[
            data(PAD), data(HIDDEN), data(HIDDEN), data(HIDDEN),
            full(PAD, HIDDEN), full(HIDDEN, SOCIAL), full(HIDDEN, 4 * HIDDEN),
            full(SOCIAL, 4 * HIDDEN), full(HIDDEN, 4 * HIDDEN),
            full(PAD, 4 * HIDDEN), full(PAD, HIDDEN), full(PAD, SOCIAL),
            full(PAD, PAD), full(HIDDEN, PAD),
        ],
        out_specs=[data(HIDDEN), data(HIDDEN), data(PAD)],
        out_shape=[
            jax.ShapeDtypeStruct((N, HIDDEN), jnp.float32),
            jax.ShapeDtypeStruct((N, HIDDEN), jnp.float32),
            jax.ShapeDtypeStruct((N, PAD), jnp.float32),
        ],
        name="social_lstm_step_tc",
    )


_tc_step = functools.cache(_make_tc_step)


def kernel(X, part_masks, all_h_t, all_c_t, Y, T_obs, T_pred, W_in, b_in,
           W_soc, b_soc, W_ih, W_hh, b_ih, b_hh, W_out, b_out):
    f32 = jnp.float32
    cid, gid = _precompute_indices(X, part_masks)
    zeros_tab = jnp.zeros((ROWS_PER_TILE, HIDDEN), f32)

    w_soc_eff = W_soc.reshape(SOCIAL, (N_SIZE + 1) ** 2, HIDDEN).sum(axis=1)
    winT = jnp.zeros((PAD, HIDDEN), f32).at[:INPUT_DIM].set(W_in.T)
    wsT = w_soc_eff.T
    wrT = W_ih[:, :MEDIATE].T
    weT = W_ih[:, MEDIATE:].T
    whhT = W_hh.T
    bg = jnp.broadcast_to(b_ih + b_hh, (PAD, 4 * HIDDEN))
    bi = jnp.broadcast_to(b_in, (PAD, HIDDEN))
    bs = jnp.broadcast_to(b_soc, (PAD, SOCIAL))
    bo = jnp.zeros((PAD, PAD), f32).at[:, :OUT_DIM].set(
        jnp.broadcast_to(b_out, (PAD, OUT_DIM)))
    woT = jnp.zeros((HIDDEN, PAD), f32).at[:, :OUT_DIM].set(W_out.T)

    Xp = jnp.zeros((NSTEP, N, PAD), f32).at[:, :, :INPUT_DIM].set(X[:NSTEP])

    h, c = all_h_t, all_c_t
    outs = []
    for t in range(NSTEP):
        inp = Xp[min(t, 9)] if t <= 9 else outs[t - 2]
        vals = _get_pool()(h, cid[t], gid[t], zeros_tab)
        h, c, out = _tc_step()(inp, h, c, vals, winT, wsT, wrT, weT, whhT,
                               bg, bi, bs, bo, woT)
        outs.append(out)
    outs.append(jnp.zeros((N, PAD), f32))
    res = jnp.stack(outs, axis=0)[:, :, :OUT_DIM]
    return res * part_masks[:, :, None]
